# baseline (speedup 1.0000x reference)
import jax
import jax.numpy as jnp
from jax.experimental import pallas as pl

N_EDGE_TYPE = 7
N_NODE_TYPE = 5
AVG_DEGREE = 7


def _matmul_body(a_ref, w_ref, o_ref):
    o_ref[...] = jax.lax.dot_general(
        a_ref[...], w_ref[...], (((1,), (0,)), ((), ())),
        preferred_element_type=jnp.float32,
        precision=jax.lax.Precision.HIGHEST,
    )


def kernel(x, edge_index, edge_type, node_type, W):
    n = x.shape[0]
    one_hot = jax.nn.one_hot(node_type, N_NODE_TYPE, dtype=x.dtype)
    xn = jnp.concatenate([x, one_hot], axis=1)
    row = edge_index[0]
    col = edge_index[1]
    gathered = jnp.take(xn, col, axis=0)
    index = row * N_EDGE_TYPE + edge_type
    agg = jax.ops.segment_sum(gathered, index, num_segments=n * N_EDGE_TYPE)
    agg = agg.reshape(n, -1)

    fan_in = W.shape[0]
    BN = 1000
    out = pl.pallas_call(
        _matmul_body,
        grid=(n // BN,),
        in_specs=[
            pl.BlockSpec((BN, fan_in), lambda i: (i, 0)),
            pl.BlockSpec((fan_in, W.shape[1]), lambda i: (0, 0)),
        ],
        out_specs=pl.BlockSpec((BN, W.shape[1]), lambda i: (i, 0)),
        out_shape=jax.ShapeDtypeStruct((n, W.shape[1]), jnp.float32),
    )(agg, W)
    return out / AVG_DEGREE


# speedup vs baseline: 3.4192x; 3.4192x over previous
import functools

import jax
import jax.numpy as jnp
from jax import lax
from jax.experimental import pallas as pl
from jax.experimental.pallas import tpu as pltpu
from jax.experimental.pallas import tpu_sc as plsc

N_EDGE_TYPE = 7
N_NODE_TYPE = 5
AVG_DEGREE = 7

BN = 1000
NP = 101000
NG = 4
WIN = 128
CHUNK = 8


def _z_body(x_ref, nt_ref, w_ref, z_ref):
    i = pl.program_id(0)
    xb = x_ref[...]
    nt = nt_ref[...]
    oh = (nt == lax.broadcasted_iota(jnp.int32, (BN, N_NODE_TYPE), 1))
    xn = jnp.concatenate(
        [xb, oh.astype(jnp.float32), jnp.zeros((BN, 3), jnp.float32)], axis=1)
    row_ids = i * BN + lax.broadcasted_iota(jnp.int32, (BN, 1), 0)
    mask = row_ids < 100000
    for t in range(N_EDGE_TYPE):
        zt = lax.dot_general(
            xn, w_ref[t], (((1,), (0,)), ((), ())),
            preferred_element_type=jnp.float32,
            precision=lax.Precision.HIGHEST)
        zt = jnp.where(mask, zt, 0.0)
        for g in range(NG):
            z_ref[g, t, :, :] = zt[:, g * 16:(g + 1) * 16]


def _compute_z(x, node_type, W):
    n = x.shape[0]
    wr = W.reshape(N_EDGE_TYPE, 37, 64) / float(AVG_DEGREE)
    wp = jnp.concatenate([wr, jnp.zeros((N_EDGE_TYPE, 3, 64), W.dtype)], axis=1)
    nt2 = node_type.reshape(n, 1).astype(jnp.int32)
    grid = (NP // BN,)
    return pl.pallas_call(
        _z_body,
        grid=grid,
        in_specs=[
            pl.BlockSpec((BN, 32), lambda i: (i, 0)),
            pl.BlockSpec((BN, 1), lambda i: (i, 0)),
            pl.BlockSpec((N_EDGE_TYPE, 40, 64), lambda i: (0, 0, 0)),
        ],
        out_specs=pl.BlockSpec((NG, N_EDGE_TYPE, BN, 16), lambda i: (0, 0, i, 0)),
        out_shape=jax.ShapeDtypeStruct((NG, N_EDGE_TYPE, NP, 16), jnp.float32),
        compiler_params=pltpu.CompilerParams(
            dimension_semantics=("arbitrary",)),
    )(x, nt2, wp)


def _sc_scatter(z4, gidx2, dst2, zeros, n_nodes, nwin):
    n_per = n_nodes // 16
    wps = nwin // 16
    nch = wps // CHUNK

    mesh = plsc.VectorSubcoreMesh(core_axis_name="c", subcore_axis_name="s")

    @functools.partial(
        pl.kernel,
        out_type=jax.ShapeDtypeStruct((NG, n_nodes, 16), jnp.float32),
        mesh=mesh,
        scratch_types=[
            pltpu.VMEM_SHARED((n_nodes, 16), jnp.float32),
            pltpu.VMEM((CHUNK, WIN), jnp.int32),
            pltpu.VMEM((CHUNK, WIN), jnp.int32),
            pltpu.VMEM((WIN, 16), jnp.float32),
            pltpu.VMEM((WIN, 16), jnp.float32),
            pltpu.SemaphoreType.DMA,
            pltpu.SemaphoreType.DMA,
        ],
        compiler_params=pltpu.CompilerParams(use_tc_tiling_on_sc=False),
    )
    def kfn(z_hbm, gidx_hbm, dst_hbm, zeros_hbm, out_hbm,
            accum, idxb, dstb, rows_a, rows_b, sem_a, sem_b):
        c = lax.axis_index("c")
        s = lax.axis_index("s")
        rows = (rows_a, rows_b)
        sems = (sem_a, sem_b)
        for gi in range(2):
            g = c * 2 + gi
            slab = z_hbm.at[g]
            out_slab = out_hbm.at[g]
            pltpu.sync_copy(zeros_hbm, accum.at[pl.ds(s * n_per, n_per)])
            plsc.subcore_barrier()

            @pl.loop(0, nch)
            def _(k):
                w0 = s * wps + k * CHUNK
                pltpu.sync_copy(gidx_hbm.at[pl.ds(w0, CHUNK)], idxb)
                pltpu.sync_copy(dst_hbm.at[pl.ds(w0, CHUNK)], dstb)
                cps = [None] * CHUNK
                cps[0] = pltpu.async_copy(slab.at[idxb.at[0]], rows[0], sems[0])
                for j in range(CHUNK):
                    if j + 1 < CHUNK:
                        cps[j + 1] = pltpu.async_copy(
                            slab.at[idxb.at[j + 1]], rows[(j + 1) % 2],
                            sems[(j + 1) % 2])
                    cps[j].wait()
                    pltpu.sync_copy(rows[j % 2], accum.at[dstb.at[j]], add=True)

            plsc.subcore_barrier()
            pltpu.sync_copy(accum.at[pl.ds(s * n_per, n_per)],
                            out_slab.at[pl.ds(s * n_per, n_per)])
            plsc.subcore_barrier()

    return kfn(z4, gidx2, dst2, zeros)


def kernel(x, edge_index, edge_type, node_type, W):
    n = x.shape[0]
    e = edge_index.shape[1]
    row = edge_index[0].astype(jnp.int32)
    col = edge_index[1].astype(jnp.int32)

    z = _compute_z(x, node_type, W)
    z4 = z.reshape(NG, N_EDGE_TYPE * NP, 16)

    gidx = edge_type.astype(jnp.int32) * NP + col
    e_pad = ((e + 16 * CHUNK * WIN - 1) // (16 * CHUNK * WIN)) * (16 * CHUNK * WIN)
    npad = e_pad - e
    pad_ar = lax.iota(jnp.int32, npad)
    gidx = jnp.concatenate([gidx, 6 * NP + n + (pad_ar % 992)])
    dst = jnp.concatenate([row, pad_ar % 4096])
    nwin = e_pad // WIN
    gidx2 = gidx.reshape(nwin, WIN)
    dst2 = dst.reshape(nwin, WIN)
    n_acc = ((n + 127) // 128) * 128
    zeros = jnp.zeros((n_acc // 16, 16), jnp.float32)

    out4 = _sc_scatter(z4, gidx2, dst2, zeros, n_acc, nwin)
    return out4[:, :n, :].transpose(1, 0, 2).reshape(n, 64)


# speedup vs baseline: 5.2642x; 1.5396x over previous
import functools

import jax
import jax.numpy as jnp
from jax import lax
from jax.experimental import pallas as pl
from jax.experimental.pallas import tpu as pltpu
from jax.experimental.pallas import tpu_sc as plsc

N_EDGE_TYPE = 7
N_NODE_TYPE = 5
AVG_DEGREE = 7

NP = 102400
NG = 4
WIN = 128
CHUNK = 8
BM = 512


def _z_body(x_ref, w_ref, z_ref):
    xb = x_ref[...]
    for s in range(NG * N_EDGE_TYPE):
        z_ref[s] = lax.dot_general(
            xb, w_ref[s], (((1,), (0,)), ((), ())),
            preferred_element_type=jnp.float32,
            precision=lax.Precision.HIGHEST)


def _compute_z(x, node_type, W):
    n = x.shape[0]
    wr = (W.reshape(N_EDGE_TYPE, 37, 64) / float(AVG_DEGREE)).astype(jnp.float32)
    wp = jnp.concatenate([wr, jnp.zeros((N_EDGE_TYPE, 3, 64), jnp.float32)], axis=1)
    wp4 = wp.reshape(N_EDGE_TYPE, 40, NG, 16)
    eye8 = jnp.eye(8, dtype=jnp.float32)
    wbig = jnp.einsum("uv,tkgj->gtukvj", eye8, wp4)
    wbig = wbig.reshape(NG * N_EDGE_TYPE, 320, 128)

    one_hot = jax.nn.one_hot(node_type, N_NODE_TYPE, dtype=jnp.float32)
    xn = jnp.concatenate([x, one_hot, jnp.zeros((n, 3), jnp.float32)], axis=1)
    xn = jnp.pad(xn, ((0, NP - n), (0, 0)))
    xn8 = xn.reshape(NP // 8, 320)

    return pl.pallas_call(
        _z_body,
        grid=(NP // 8 // BM,),
        in_specs=[
            pl.BlockSpec((BM, 320), lambda i: (i, 0)),
            pl.BlockSpec((NG * N_EDGE_TYPE, 320, 128), lambda i: (0, 0, 0)),
        ],
        out_specs=pl.BlockSpec((NG * N_EDGE_TYPE, BM, 128), lambda i: (0, i, 0)),
        out_shape=jax.ShapeDtypeStruct((NG * N_EDGE_TYPE, NP // 8, 128),
                                       jnp.float32),
        compiler_params=pltpu.CompilerParams(
            dimension_semantics=("arbitrary",)),
    )(xn8, wbig)


def _sc_scatter(z4, gidx2, dst2, zeros, n_nodes, nwin):
    n_per = n_nodes // 16
    wps = nwin // 16
    nch = wps // CHUNK

    mesh = plsc.VectorSubcoreMesh(core_axis_name="c", subcore_axis_name="s")

    @functools.partial(
        pl.kernel,
        out_type=jax.ShapeDtypeStruct((NG, n_nodes, 16), jnp.float32),
        mesh=mesh,
        scratch_types=[
            pltpu.VMEM_SHARED((n_nodes, 16), jnp.float32),
            pltpu.VMEM((CHUNK, WIN), jnp.int32),
            pltpu.VMEM((CHUNK, WIN), jnp.int32),
            pltpu.VMEM((WIN, 16), jnp.float32),
            pltpu.VMEM((WIN, 16), jnp.float32),
            pltpu.SemaphoreType.DMA,
            pltpu.SemaphoreType.DMA,
        ],
        compiler_params=pltpu.CompilerParams(use_tc_tiling_on_sc=False),
    )
    def kfn(z_hbm, gidx_hbm, dst_hbm, zeros_hbm, out_hbm,
            accum, idxb, dstb, rows_a, rows_b, sem_a, sem_b):
        c = lax.axis_index("c")
        s = lax.axis_index("s")
        rows = (rows_a, rows_b)
        sems = (sem_a, sem_b)
        for gi in range(2):
            g = c * 2 + gi
            slab = z_hbm.at[g]
            out_slab = out_hbm.at[g]
            pltpu.sync_copy(zeros_hbm, accum.at[pl.ds(s * n_per, n_per)])
            plsc.subcore_barrier()

            @pl.loop(0, nch)
            def _(k):
                w0 = s * wps + k * CHUNK
                pltpu.sync_copy(gidx_hbm.at[pl.ds(w0, CHUNK)], idxb)
                pltpu.sync_copy(dst_hbm.at[pl.ds(w0, CHUNK)], dstb)
                cps = [None] * CHUNK
                cps[0] = pltpu.async_copy(slab.at[idxb.at[0]], rows[0], sems[0])
                for j in range(CHUNK):
                    if j + 1 < CHUNK:
                        cps[j + 1] = pltpu.async_copy(
                            slab.at[idxb.at[j + 1]], rows[(j + 1) % 2],
                            sems[(j + 1) % 2])
                    cps[j].wait()
                    pltpu.sync_copy(rows[j % 2], accum.at[dstb.at[j]], add=True)

            plsc.subcore_barrier()
            pltpu.sync_copy(accum.at[pl.ds(s * n_per, n_per)],
                            out_slab.at[pl.ds(s * n_per, n_per)])
            plsc.subcore_barrier()

    return kfn(z4, gidx2, dst2, zeros)


def kernel(x, edge_index, edge_type, node_type, W):
    n = x.shape[0]
    e = edge_index.shape[1]
    row = edge_index[0].astype(jnp.int32)
    col = edge_index[1].astype(jnp.int32)

    z = _compute_z(x, node_type, W)
    z4 = z.reshape(NG, N_EDGE_TYPE * NP, 16)

    gidx = edge_type.astype(jnp.int32) * NP + col
    e_pad = ((e + 16 * CHUNK * WIN - 1) // (16 * CHUNK * WIN)) * (16 * CHUNK * WIN)
    npad = e_pad - e
    pad_ar = lax.iota(jnp.int32, npad)
    gidx = jnp.concatenate([gidx, 6 * NP + n + (pad_ar % 992)])
    dst = jnp.concatenate([row, pad_ar % 4096])
    nwin = e_pad // WIN
    gidx2 = gidx.reshape(nwin, WIN)
    dst2 = dst.reshape(nwin, WIN)
    n_acc = ((n + 127) // 128) * 128
    zeros = jnp.zeros((n_acc // 16, 16), jnp.float32)

    out4 = _sc_scatter(z4, gidx2, dst2, zeros, n_acc, nwin)
    return out4[:, :n, :].transpose(1, 0, 2).reshape(n, 64)


# speedup vs baseline: 7.1615x; 1.3604x over previous
import functools

import jax
import jax.numpy as jnp
from jax import lax
from jax.experimental import pallas as pl
from jax.experimental.pallas import tpu as pltpu
from jax.experimental.pallas import tpu_sc as plsc

N_EDGE_TYPE = 7
N_NODE_TYPE = 5
AVG_DEGREE = 7

NP = 102400
NG = 4
WIN = 128
CHUNK = 8
BM = 512


def _z_body(x_ref, w_ref, z_ref):
    xb = x_ref[...]
    for s in range(NG * N_EDGE_TYPE):
        z_ref[s] = lax.dot_general(
            xb, w_ref[s], (((1,), (0,)), ((), ())),
            preferred_element_type=jnp.float32,
            precision=lax.Precision.HIGHEST)


def _compute_z(x, node_type, W):
    n = x.shape[0]
    wr = (W.reshape(N_EDGE_TYPE, 37, 64) / float(AVG_DEGREE)).astype(jnp.float32)
    wp = jnp.concatenate([wr, jnp.zeros((N_EDGE_TYPE, 3, 64), jnp.float32)], axis=1)
    wp4 = wp.reshape(N_EDGE_TYPE, 40, NG, 16)
    eye8 = jnp.eye(8, dtype=jnp.float32)
    wbig = jnp.einsum("uv,tkgj->gtukvj", eye8, wp4)
    wbig = wbig.reshape(NG * N_EDGE_TYPE, 320, 128)

    one_hot = jax.nn.one_hot(node_type, N_NODE_TYPE, dtype=jnp.float32)
    xn = jnp.concatenate([x, one_hot, jnp.zeros((n, 3), jnp.float32)], axis=1)
    xn = jnp.pad(xn, ((0, NP - n), (0, 0)))
    xn8 = xn.reshape(NP // 8, 320)

    return pl.pallas_call(
        _z_body,
        grid=(NP // 8 // BM,),
        in_specs=[
            pl.BlockSpec((BM, 320), lambda i: (i, 0)),
            pl.BlockSpec((NG * N_EDGE_TYPE, 320, 128), lambda i: (0, 0, 0)),
        ],
        out_specs=pl.BlockSpec((NG * N_EDGE_TYPE, BM, 128), lambda i: (0, i, 0)),
        out_shape=jax.ShapeDtypeStruct((NG * N_EDGE_TYPE, NP // 8, 128),
                                       jnp.float32),
        compiler_params=pltpu.CompilerParams(
            dimension_semantics=("arbitrary",)),
    )(xn8, wbig)


def _sc_scatter(z4, gidx2, dst2, zeros, n_nodes, nwin):
    n_per = n_nodes // 16
    wps = nwin // 16
    nsc = wps // (7 * CHUNK)
    NBUF = 8

    mesh = plsc.VectorSubcoreMesh(core_axis_name="c", subcore_axis_name="s")

    row_bufs = [pltpu.VMEM((WIN, 16), jnp.float32) for _ in range(NBUF)]
    gsems = [pltpu.SemaphoreType.DMA for _ in range(NBUF)]
    ssems = [pltpu.SemaphoreType.DMA for _ in range(NBUF)]

    @functools.partial(
        pl.kernel,
        out_type=jax.ShapeDtypeStruct((NG, n_nodes, 16), jnp.float32),
        mesh=mesh,
        scratch_types=[
            pltpu.VMEM_SHARED((n_nodes, 16), jnp.float32),
            pltpu.VMEM((7 * CHUNK, WIN), jnp.int32),
            pltpu.VMEM((7 * CHUNK, WIN), jnp.int32),
        ] + row_bufs + gsems + ssems,
        compiler_params=pltpu.CompilerParams(use_tc_tiling_on_sc=False),
    )
    def kfn(z_hbm, gidx_hbm, dst_hbm, zeros_hbm, out_hbm,
            accum, idxb, dstb, *bufs_and_sems):
        rows = bufs_and_sems[:NBUF]
        gsem = bufs_and_sems[NBUF:2 * NBUF]
        ssem = bufs_and_sems[2 * NBUF:3 * NBUF]
        c = lax.axis_index("c")
        s = lax.axis_index("s")
        for gi in range(2):
            g = c * 2 + gi
            slab = z_hbm.at[g]
            out_slab = out_hbm.at[g]
            pltpu.sync_copy(zeros_hbm, accum.at[pl.ds(s * n_per, n_per)])
            plsc.subcore_barrier()

            @pl.loop(0, nsc)
            def _(sk):
                w0 = s * wps + sk * (7 * CHUNK)
                pltpu.sync_copy(gidx_hbm.at[pl.ds(w0, 7 * CHUNK)], idxb)
                pltpu.sync_copy(dst_hbm.at[pl.ds(w0, 7 * CHUNK)], dstb)

                @pl.loop(0, 7)
                def _(ck):
                    b = ck * CHUNK
                    cps = [None] * CHUNK
                    scps = [None] * CHUNK
                    for j in range(4):
                        cps[j] = pltpu.async_copy(
                            slab.at[idxb.at[b + j]], rows[j], gsem[j])
                    for j in range(CHUNK):
                        if j + 4 < CHUNK:
                            cps[j + 4] = pltpu.async_copy(
                                slab.at[idxb.at[b + j + 4]], rows[j + 4],
                                gsem[j + 4])
                        cps[j].wait()
                        scps[j] = pltpu.async_copy(
                            rows[j], accum.at[dstb.at[b + j]], ssem[j],
                            add=True)
                    for j in range(CHUNK):
                        scps[j].wait()

            plsc.subcore_barrier()
            pltpu.sync_copy(accum.at[pl.ds(s * n_per, n_per)],
                            out_slab.at[pl.ds(s * n_per, n_per)])
            plsc.subcore_barrier()

    return kfn(z4, gidx2, dst2, zeros)


def kernel(x, edge_index, edge_type, node_type, W):
    n = x.shape[0]
    e = edge_index.shape[1]
    row = edge_index[0].astype(jnp.int32)
    col = edge_index[1].astype(jnp.int32)

    z = _compute_z(x, node_type, W)
    z4 = z.reshape(NG, N_EDGE_TYPE * NP, 16)

    gidx = edge_type.astype(jnp.int32) * NP + col
    quant = 16 * 7 * CHUNK * WIN
    e_pad = ((e + quant - 1) // quant) * quant
    npad = e_pad - e
    pad_ar = lax.iota(jnp.int32, npad)
    gidx = jnp.concatenate([gidx, 6 * NP + n + (pad_ar % 992)])
    dst = jnp.concatenate([row, pad_ar % 4096])
    nwin = e_pad // WIN
    gidx2 = gidx.reshape(nwin, WIN)
    dst2 = dst.reshape(nwin, WIN)
    n_acc = ((n + 127) // 128) * 128
    zeros = jnp.zeros((n_acc // 16, 16), jnp.float32)

    out4 = _sc_scatter(z4, gidx2, dst2, zeros, n_acc, nwin)
    return out4[:, :n, :].transpose(1, 0, 2).reshape(n, 64)


# speedup vs baseline: 8.3275x; 1.1628x over previous
import functools

import jax
import jax.numpy as jnp
from jax import lax
from jax.experimental import pallas as pl
from jax.experimental.pallas import tpu as pltpu
from jax.experimental.pallas import tpu_sc as plsc

N_EDGE_TYPE = 7
N_NODE_TYPE = 5
AVG_DEGREE = 7

NP = 102000
NG = 4
NS = 32
WIN = 128
CHUNK = 8
BN = 2000


def _z_body(x_ref, nt_ref, w_ref, z_ref):
    i = pl.program_id(0)
    xb = x_ref[...]
    nt = nt_ref[...]
    oh = (nt == lax.broadcasted_iota(jnp.int32, (BN, N_NODE_TYPE), 1))
    xn = jnp.concatenate(
        [xb, oh.astype(jnp.float32), jnp.zeros((BN, 3), jnp.float32)], axis=1)
    z = lax.dot_general(
        xn, w_ref[...], (((1,), (0,)), ((), ())),
        preferred_element_type=jnp.float32,
        precision=lax.Precision.HIGHEST)
    row_ids = i * BN + lax.broadcasted_iota(jnp.int32, (BN, 1), 0)
    z_ref[...] = jnp.where(row_ids < 100000, z, 0.0)


def _compute_z(x, node_type, W):
    n = x.shape[0]
    wr = (W.reshape(N_EDGE_TYPE, 37, 64) / float(AVG_DEGREE)).astype(jnp.float32)
    wp = jnp.concatenate([wr, jnp.zeros((N_EDGE_TYPE, 3, 64), jnp.float32)], axis=1)
    wq = wp.reshape(N_EDGE_TYPE, 40, NG, 16).transpose(1, 2, 0, 3).reshape(40, 448)
    wq = jnp.pad(wq, ((0, 0), (0, 64)))
    nt2 = node_type.reshape(n, 1).astype(jnp.int32)

    return pl.pallas_call(
        _z_body,
        grid=(NP // BN,),
        in_specs=[
            pl.BlockSpec((BN, 32), lambda i: (i, 0)),
            pl.BlockSpec((BN, 1), lambda i: (i, 0)),
            pl.BlockSpec((40, 512), lambda i: (0, 0)),
        ],
        out_specs=pl.BlockSpec((BN, 512), lambda i: (i, 0)),
        out_shape=jax.ShapeDtypeStruct((NP, 512), jnp.float32),
        compiler_params=pltpu.CompilerParams(
            dimension_semantics=("arbitrary",)),
    )(x, nt2, wq)


def _sc_scatter(z16, gidx2, dst2, zeros, n_nodes, nwin):
    n_per = n_nodes // 16
    wps = nwin // 16
    nsc = wps // (7 * CHUNK)
    NBUF = 8

    mesh = plsc.VectorSubcoreMesh(core_axis_name="c", subcore_axis_name="s")

    row_bufs = [pltpu.VMEM((WIN, 16), jnp.float32) for _ in range(NBUF)]
    gsems = [pltpu.SemaphoreType.DMA for _ in range(NBUF)]
    ssems = [pltpu.SemaphoreType.DMA for _ in range(NBUF)]

    @functools.partial(
        pl.kernel,
        out_type=jax.ShapeDtypeStruct((NG, n_nodes, 16), jnp.float32),
        mesh=mesh,
        scratch_types=[
            pltpu.VMEM_SHARED((n_nodes, 16), jnp.float32),
            pltpu.VMEM((7 * CHUNK, WIN), jnp.int32),
            pltpu.VMEM((7 * CHUNK, WIN), jnp.int32),
        ] + row_bufs + gsems + ssems,
        compiler_params=pltpu.CompilerParams(use_tc_tiling_on_sc=False),
    )
    def kfn(z_hbm, gidx_hbm, dst_hbm, zeros_hbm, out_hbm,
            accum, idxb, dstb, *bufs_and_sems):
        rows = bufs_and_sems[:NBUF]
        gsem = bufs_and_sems[NBUF:2 * NBUF]
        ssem = bufs_and_sems[2 * NBUF:3 * NBUF]
        c = lax.axis_index("c")
        s = lax.axis_index("s")
        for gi in range(2):
            g = c * 2 + gi
            off = g * N_EDGE_TYPE
            slab = z_hbm.at[pl.ds(off, NP * NS - NS + 4)]
            out_slab = out_hbm.at[g]
            pltpu.sync_copy(zeros_hbm, accum.at[pl.ds(s * n_per, n_per)])
            plsc.subcore_barrier()

            @pl.loop(0, nsc)
            def _(sk):
                w0 = s * wps + sk * (7 * CHUNK)
                pltpu.sync_copy(gidx_hbm.at[pl.ds(w0, 7 * CHUNK)], idxb)
                pltpu.sync_copy(dst_hbm.at[pl.ds(w0, 7 * CHUNK)], dstb)

                @pl.loop(0, 7)
                def _(ck):
                    b = ck * CHUNK
                    cps = [None] * CHUNK
                    scps = [None] * CHUNK
                    for j in range(4):
                        cps[j] = pltpu.async_copy(
                            slab.at[idxb.at[b + j]], rows[j], gsem[j])
                    for j in range(CHUNK):
                        if j + 4 < CHUNK:
                            cps[j + 4] = pltpu.async_copy(
                                slab.at[idxb.at[b + j + 4]], rows[j + 4],
                                gsem[j + 4])
                        cps[j].wait()
                        scps[j] = pltpu.async_copy(
                            rows[j], accum.at[dstb.at[b + j]], ssem[j],
                            add=True)
                    for j in range(CHUNK):
                        scps[j].wait()

            plsc.subcore_barrier()
            pltpu.sync_copy(accum.at[pl.ds(s * n_per, n_per)],
                            out_slab.at[pl.ds(s * n_per, n_per)])
            plsc.subcore_barrier()

    return kfn(z16, gidx2, dst2, zeros)


def kernel(x, edge_index, edge_type, node_type, W):
    n = x.shape[0]
    e = edge_index.shape[1]
    row = edge_index[0].astype(jnp.int32)
    col = edge_index[1].astype(jnp.int32)

    z = _compute_z(x, node_type, W)
    z16 = z.reshape(NP * NS, 16)

    gidx = col * NS + edge_type.astype(jnp.int32)
    quant = 16 * 7 * CHUNK * WIN
    e_pad = ((e + quant - 1) // quant) * quant
    npad = e_pad - e
    pad_ar = lax.iota(jnp.int32, npad)
    gidx = jnp.concatenate([gidx, (n + (pad_ar % 992)) * NS])
    dst = jnp.concatenate([row, pad_ar % 4096])
    nwin = e_pad // WIN
    gidx2 = gidx.reshape(nwin, WIN)
    dst2 = dst.reshape(nwin, WIN)
    n_acc = ((n + 127) // 128) * 128
    zeros = jnp.zeros((n_acc // 16, 16), jnp.float32)

    out4 = _sc_scatter(z16, gidx2, dst2, zeros, n_acc, nwin)
    return out4[:, :n, :].transpose(1, 0, 2).reshape(n, 64)
